# baseline (speedup 1.0000x reference)
import functools
import math

import jax
import jax.numpy as jnp
from jax import lax
from jax.experimental import pallas as pl
from jax.experimental.pallas import tpu as pltpu
from jax.experimental.pallas import tpu_sc as plsc

NC = 2
NS = 16
L = 16
NP = 32


def _make_sc_kernel(B, N, NPTS):
    NW = NC * NS
    BPW = B // NW
    NGR = BPW // L
    FLAT = BPW * NP
    NSQ = NPTS * NPTS
    NROW = FLAT // BPW

    mesh = plsc.VectorSubcoreMesh(core_axis_name="c", subcore_axis_name="s",
                                  num_cores=NC, num_subcores=NS)

    @functools.partial(
        pl.kernel,
        out_type=(
            jax.ShapeDtypeStruct((B,), jnp.float32),
            jax.ShapeDtypeStruct((B,), jnp.float32),
            jax.ShapeDtypeStruct((B,), jnp.float32),
        ),
        mesh=mesh,
        compiler_params=pltpu.CompilerParams(needs_layout_passes=False,
                                             use_tc_tiling_on_sc=False),
        scratch_types=[
            pltpu.VMEM((BPW,), jnp.int32),
            pltpu.VMEM((BPW, NP), jnp.int32),
            pltpu.VMEM((NROW, BPW), jnp.int32),
            pltpu.VMEM((FLAT,), jnp.float32),
            pltpu.VMEM((BPW * NSQ,), jnp.float32),
            pltpu.VMEM((BPW,), jnp.float32),
            pltpu.VMEM((BPW,), jnp.float32),
            pltpu.VMEM((BPW,), jnp.float32),
            pltpu.VMEM((BPW,), jnp.float32),
            pltpu.SemaphoreType.DMA,
            pltpu.SemaphoreType.DMA,
            pltpu.SemaphoreType.DMA,
            pltpu.SemaphoreType.DMA,
        ],
    )
    def sc_kernel(idx_h, scores_h, slkp_h, nbr_h, w_h, mean_h, std_h,
                  m_out, locs_out, scales_out,
                  idx_v, nbr_raw, nbr2d, ns_flat, w_flat, s_v, locs_v,
                  scales_v, m_v, sem_w, sem_n, sem_ls, sem_s):
        wid = lax.axis_index("s") * NC + lax.axis_index("c")
        base = wid * BPW

        pltpu.sync_copy(idx_h.at[pl.ds(base, BPW)], idx_v)
        cl = pltpu.async_copy(mean_h.at[idx_v], locs_v, sem_ls)
        cs = pltpu.async_copy(std_h.at[idx_v], scales_v, sem_ls)
        pltpu.sync_copy(scores_h.at[pl.ds(base, BPW)], s_v)

        cw = pltpu.async_copy(w_h.at[pl.ds(base * NSQ, BPW * NSQ)], w_flat,
                              sem_w)
        n_chunks = []
        for c in range(BPW // L):
            ivec = idx_v[pl.ds(c * L, L)]
            chunk = []
            for u in range(L):
                e = c * L + u
                iu = ivec[u]
                chunk.append(pltpu.async_copy(
                    nbr_h.at[pl.ds(iu * NP, NP)],
                    nbr_raw.at[e], sem_n))
            n_chunks.append(chunk)
            if c > 0:
                for cp in n_chunks[c - 1]:
                    cp.wait()
        for cp in n_chunks[-1]:
            cp.wait()
        for t in range(NROW):
            for g in range(BPW // L):
                p = t * BPW + g * L
                v = nbr_raw[p // NP, pl.ds(p % NP, L)]
                nbr2d[t, pl.ds(g * L, L)] = jnp.where(v < 0, N, v)

        ns_copies = []
        for t in range(NROW):
            ns_copies.append(pltpu.async_copy(
                slkp_h.at[nbr2d.at[t]],
                ns_flat.at[pl.ds(t * BPW, BPW)], sem_s))

        for cp in ns_copies:
            cp.wait()
        cw.wait()
        cl.wait()
        cs.wait()

        iota = lax.broadcasted_iota(jnp.int32, (L,), 0)

        def group_body(g, carry):
            e = g * L
            erow = e + iota
            nsbase = erow * NP
            cols = [s_v[pl.ds(e, L)]]
            for i in range(1, NPTS):
                cols.append(plsc.load_gather(ns_flat, [nsbase + i]))
            s = cols[0]
            for i in range(1, NPTS):
                s = s + cols[i]
            mean = s * (1.0 / NPTS)
            X = [c - mean for c in cols]
            wbase = erow * NSQ
            m = jnp.zeros((L,), jnp.float32)
            for i in range(NPTS):
                y = jnp.zeros((L,), jnp.float32)
                for j in range(NPTS):
                    wv = plsc.load_gather(w_flat, [wbase + (i * NPTS + j)])
                    y = y + wv * X[j]
                m = m + X[i] * y
            m_v[pl.ds(e, L)] = m
            return carry

        lax.fori_loop(0, NGR, group_body, 0)

        pltpu.sync_copy(m_v, m_out.at[pl.ds(base, BPW)])
        pltpu.sync_copy(locs_v, locs_out.at[pl.ds(base, BPW)])
        pltpu.sync_copy(scales_v, scales_out.at[pl.ds(base, BPW)])

    return sc_kernel


def _make_w_bounce_kernel(B, N, NPTS):
    NW = NC * NS
    BPW = B // NW
    mesh = plsc.VectorSubcoreMesh(core_axis_name="c", subcore_axis_name="s",
                                  num_cores=NC, num_subcores=NS)

    @functools.partial(
        pl.kernel,
        out_type=jax.ShapeDtypeStruct((B, NPTS, NPTS), jnp.float32),
        mesh=mesh,
        compiler_params=pltpu.CompilerParams(needs_layout_passes=False,
                                             use_tc_tiling_on_sc=True),
        scratch_types=[
            pltpu.VMEM((BPW,), jnp.int32),
            pltpu.SemaphoreType.DMA,
        ],
    )
    def w_bounce(idx_h, w_h, w_out, idx_v, sem):
        wid = lax.axis_index("s") * NC + lax.axis_index("c")
        base = wid * BPW
        pltpu.sync_copy(idx_h.at[pl.ds(base, BPW)], idx_v)
        chunks = []
        for c in range(BPW // L):
            ivec = idx_v[pl.ds(c * L, L)]
            chunk = [pltpu.async_copy(w_h.at[ivec[u]],
                                      w_out.at[base + c * L + u], sem)
                     for u in range(L)]
            chunks.append(chunk)
            if c > 0:
                for cp in chunks[c - 1]:
                    cp.wait()
        for cp in chunks[-1]:
            cp.wait()

    return w_bounce


def _tc_tail_body(m_ref, locs_ref, scales_ref, o_ref):
    m = m_ref[...]
    locs = locs_ref[...]
    scales = scales_ref[...]
    lmin = jnp.min(locs)
    left = jnp.minimum(m, 2.0 * lmin - m)
    z = jnp.clip((left - locs) / scales, -5.0, 0.0)
    cdf2 = 1.0 + lax.erf(z * (1.0 / math.sqrt(2.0)))
    o_ref[...] = -jnp.log(cdf2 + 1e-12)


@jax.jit
def kernel(idx, scores, scores_lookup, neighborhood_scores_idx_lookup,
           weight_matrix_lookup, mean_lookup, std_lookup):
    B = idx.shape[0]
    N = scores_lookup.shape[0] - 1
    NPTS = neighborhood_scores_idx_lookup.shape[1]

    nbr_pad = jnp.pad(neighborhood_scores_idx_lookup,
                      ((0, 0), (0, NP - NPTS))).reshape(N * NP)
    w_g = _make_w_bounce_kernel(B, N, NPTS)(idx, weight_matrix_lookup)
    w_rows = w_g.reshape(B * NPTS * NPTS)

    sc_k = _make_sc_kernel(B, N, NPTS)
    m, locs, scales = sc_k(idx, scores, scores_lookup, nbr_pad, w_rows,
                           mean_lookup, std_lookup)

    rows = B // 128
    out = pl.pallas_call(
        _tc_tail_body,
        out_shape=jax.ShapeDtypeStruct((rows, 128), jnp.float32),
    )(m.reshape(rows, 128), locs.reshape(rows, 128), scales.reshape(rows, 128))
    return out.reshape(B)


# speedup vs baseline: 2.2581x; 2.2581x over previous
import functools
import math

import jax
import jax.numpy as jnp
from jax import lax
from jax.experimental import pallas as pl
from jax.experimental.pallas import tpu as pltpu
from jax.experimental.pallas import tpu_sc as plsc

NC = 2
NS = 16
L = 16
NP = 32


def _make_sc_kernel(B, N, NPTS):
    NW = NC * NS
    BPW = B // NW
    NGR = BPW // L
    FLAT = BPW * NP
    NSQ = NPTS * NPTS
    NROW = FLAT // BPW

    mesh = plsc.VectorSubcoreMesh(core_axis_name="c", subcore_axis_name="s",
                                  num_cores=NC, num_subcores=NS)

    @functools.partial(
        pl.kernel,
        out_type=(
            jax.ShapeDtypeStruct((B,), jnp.float32),
            jax.ShapeDtypeStruct((B,), jnp.float32),
            jax.ShapeDtypeStruct((B,), jnp.float32),
        ),
        mesh=mesh,
        compiler_params=pltpu.CompilerParams(needs_layout_passes=False,
                                             use_tc_tiling_on_sc=False),
        scratch_types=[
            pltpu.VMEM((BPW,), jnp.int32),
            pltpu.VMEM((BPW, NP), jnp.int32),
            pltpu.VMEM((NROW, BPW), jnp.int32),
            pltpu.VMEM((FLAT,), jnp.float32),
            pltpu.VMEM((BPW * NSQ,), jnp.float32),
            pltpu.VMEM((BPW,), jnp.float32),
            pltpu.VMEM((BPW,), jnp.float32),
            pltpu.VMEM((BPW,), jnp.float32),
            pltpu.VMEM((BPW,), jnp.float32),
            pltpu.SemaphoreType.DMA,
            pltpu.SemaphoreType.DMA,
            pltpu.SemaphoreType.DMA,
            pltpu.SemaphoreType.DMA,
        ],
    )
    def sc_kernel(idx_h, scores_h, slkp_h, nbr_h, w_h, mean_h, std_h,
                  m_out, locs_out, scales_out,
                  idx_v, nbr_raw, nbr2d, ns_flat, w_flat, s_v, locs_v,
                  scales_v, m_v, sem_w, sem_n, sem_ls, sem_s):
        wid = lax.axis_index("s") * NC + lax.axis_index("c")
        base = wid * BPW

        pltpu.sync_copy(idx_h.at[pl.ds(base, BPW)], idx_v)
        cl = pltpu.async_copy(mean_h.at[idx_v], locs_v, sem_ls)
        cs = pltpu.async_copy(std_h.at[idx_v], scales_v, sem_ls)
        pltpu.sync_copy(scores_h.at[pl.ds(base, BPW)], s_v)

        cw = pltpu.async_copy(w_h.at[pl.ds(base * NSQ, BPW * NSQ)], w_flat,
                              sem_w)
        n_chunks = []
        for c in range(BPW // L):
            ivec = idx_v[pl.ds(c * L, L)]
            chunk = []
            for u in range(L):
                e = c * L + u
                iu = ivec[u]
                chunk.append(pltpu.async_copy(
                    nbr_h.at[pl.ds(iu * NP, NP)],
                    nbr_raw.at[e], sem_n))
            n_chunks.append(chunk)
            if c > 0:
                for cp in n_chunks[c - 1]:
                    cp.wait()
        for cp in n_chunks[-1]:
            cp.wait()
        for t in range(NROW):
            for g in range(BPW // L):
                p = t * BPW + g * L
                v = nbr_raw[p // NP, pl.ds(p % NP, L)]
                nbr2d[t, pl.ds(g * L, L)] = jnp.where(v < 0, N, v)

        ns_copies = []
        for t in range(NROW):
            ns_copies.append(pltpu.async_copy(
                slkp_h.at[nbr2d.at[t]],
                ns_flat.at[pl.ds(t * BPW, BPW)], sem_s))

        for cp in ns_copies:
            cp.wait()
        cw.wait()
        cl.wait()
        cs.wait()

        iota = lax.broadcasted_iota(jnp.int32, (L,), 0)

        def group_body(g, carry):
            e = g * L
            erow = e + iota
            nsbase = erow * NP
            cols = [s_v[pl.ds(e, L)]]
            for i in range(1, NPTS):
                cols.append(plsc.load_gather(ns_flat, [nsbase + i]))
            s = cols[0]
            for i in range(1, NPTS):
                s = s + cols[i]
            mean = s * (1.0 / NPTS)
            X = [c - mean for c in cols]
            wbase = erow * NSQ
            m = jnp.zeros((L,), jnp.float32)
            for i in range(NPTS):
                y = jnp.zeros((L,), jnp.float32)
                for j in range(NPTS):
                    wv = plsc.load_gather(w_flat, [wbase + (i * NPTS + j)])
                    y = y + wv * X[j]
                m = m + X[i] * y
            m_v[pl.ds(e, L)] = m
            return carry

        lax.fori_loop(0, NGR, group_body, 0)

        pltpu.sync_copy(m_v, m_out.at[pl.ds(base, BPW)])
        pltpu.sync_copy(locs_v, locs_out.at[pl.ds(base, BPW)])
        pltpu.sync_copy(scales_v, scales_out.at[pl.ds(base, BPW)])

    return sc_kernel


def _make_w_gather_tc(B, N, NPTS):
    BLK = 128
    NB = B // BLK

    def body(idx_ref, w_any, o_ref, sem):
        g = pl.program_id(0)
        cps = []
        for u in range(BLK):
            iv = idx_ref[g * BLK + u]
            cp = pltpu.make_async_copy(w_any.at[iv], o_ref.at[u], sem)
            cp.start()
            cps.append(cp)
        for cp in cps:
            cp.wait()

    grid_spec = pltpu.PrefetchScalarGridSpec(
        num_scalar_prefetch=1,
        grid=(NB,),
        in_specs=[pl.BlockSpec(memory_space=pl.ANY)],
        out_specs=pl.BlockSpec((BLK, NPTS, NPTS), lambda g, idx_ref: (g, 0, 0)),
        scratch_shapes=[pltpu.SemaphoreType.DMA],
    )
    return pl.pallas_call(
        body,
        grid_spec=grid_spec,
        out_shape=jax.ShapeDtypeStruct((B, NPTS, NPTS), jnp.float32),
    )


def _tc_tail_body(m_ref, locs_ref, scales_ref, o_ref):
    m = m_ref[...]
    locs = locs_ref[...]
    scales = scales_ref[...]
    lmin = jnp.min(locs)
    left = jnp.minimum(m, 2.0 * lmin - m)
    z = jnp.clip((left - locs) / scales, -5.0, 0.0)
    cdf2 = 1.0 + lax.erf(z * (1.0 / math.sqrt(2.0)))
    o_ref[...] = -jnp.log(cdf2 + 1e-12)


@jax.jit
def kernel(idx, scores, scores_lookup, neighborhood_scores_idx_lookup,
           weight_matrix_lookup, mean_lookup, std_lookup):
    B = idx.shape[0]
    N = scores_lookup.shape[0] - 1
    NPTS = neighborhood_scores_idx_lookup.shape[1]

    nbr_pad = jnp.pad(neighborhood_scores_idx_lookup,
                      ((0, 0), (0, NP - NPTS))).reshape(N * NP)
    w_g = _make_w_gather_tc(B, N, NPTS)(idx, weight_matrix_lookup)
    w_rows = w_g.reshape(B * NPTS * NPTS)

    sc_k = _make_sc_kernel(B, N, NPTS)
    m, locs, scales = sc_k(idx, scores, scores_lookup, nbr_pad, w_rows,
                           mean_lookup, std_lookup)

    rows = B // 128
    out = pl.pallas_call(
        _tc_tail_body,
        out_shape=jax.ShapeDtypeStruct((rows, 128), jnp.float32),
    )(m.reshape(rows, 128), locs.reshape(rows, 128), scales.reshape(rows, 128))
    return out.reshape(B)


# speedup vs baseline: 2.2585x; 1.0002x over previous
import functools
import math

import jax
import jax.numpy as jnp
from jax import lax
from jax.experimental import pallas as pl
from jax.experimental.pallas import tpu as pltpu
from jax.experimental.pallas import tpu_sc as plsc

NC = 2
NS = 16
L = 16
NP = 32


def _make_sc_kernel(B, N, NPTS):
    NW = NC * NS
    BPW = B // NW
    NGR = BPW // L
    FLAT = BPW * NP
    NSQ = NPTS * NPTS
    NROW = FLAT // BPW

    mesh = plsc.VectorSubcoreMesh(core_axis_name="c", subcore_axis_name="s",
                                  num_cores=NC, num_subcores=NS)

    @functools.partial(
        pl.kernel,
        out_type=(
            jax.ShapeDtypeStruct((B,), jnp.float32),
            jax.ShapeDtypeStruct((B,), jnp.float32),
            jax.ShapeDtypeStruct((B,), jnp.float32),
        ),
        mesh=mesh,
        compiler_params=pltpu.CompilerParams(needs_layout_passes=False,
                                             use_tc_tiling_on_sc=False),
        scratch_types=[
            pltpu.VMEM((BPW,), jnp.int32),
            pltpu.VMEM((BPW, NP), jnp.int32),
            pltpu.VMEM((NROW, BPW), jnp.int32),
            pltpu.VMEM((FLAT,), jnp.float32),
            pltpu.VMEM((BPW * NSQ,), jnp.float32),
            pltpu.VMEM((BPW,), jnp.float32),
            pltpu.VMEM((BPW,), jnp.float32),
            pltpu.VMEM((BPW,), jnp.float32),
            pltpu.VMEM((BPW,), jnp.float32),
            pltpu.SemaphoreType.DMA,
            pltpu.SemaphoreType.DMA,
            pltpu.SemaphoreType.DMA,
            pltpu.SemaphoreType.DMA,
        ],
    )
    def sc_kernel(idx_h, scores_h, slkp_h, nbr_h, w_h, mean_h, std_h,
                  m_out, locs_out, scales_out,
                  idx_v, nbr_raw, nbr2d, ns_flat, w_flat, s_v, locs_v,
                  scales_v, m_v, sem_w, sem_n, sem_ls, sem_s):
        wid = lax.axis_index("s") * NC + lax.axis_index("c")
        base = wid * BPW

        pltpu.sync_copy(idx_h.at[pl.ds(base, BPW)], idx_v)
        scope = jax.named_scope
        cl = pltpu.async_copy(mean_h.at[idx_v], locs_v, sem_ls)
        cs = pltpu.async_copy(std_h.at[idx_v], scales_v, sem_ls)
        pltpu.sync_copy(scores_h.at[pl.ds(base, BPW)], s_v)

        cw = pltpu.async_copy(w_h.at[pl.ds(base * NSQ, BPW * NSQ)], w_flat,
                              sem_w)
        scope_n = scope("nbr_dma"); scope_n.__enter__()
        n_chunks = []
        for c in range(BPW // L):
            ivec = idx_v[pl.ds(c * L, L)]
            chunk = []
            for u in range(L):
                e = c * L + u
                iu = ivec[u]
                chunk.append(pltpu.async_copy(
                    nbr_h.at[pl.ds(iu * NP, NP)],
                    nbr_raw.at[e], sem_n))
            n_chunks.append(chunk)
            if c > 0:
                for cp in n_chunks[c - 1]:
                    cp.wait()
        for cp in n_chunks[-1]:
            cp.wait()
        scope_n.__exit__(None, None, None)
        scope_f = scope("fixstage"); scope_f.__enter__()
        for t in range(NROW):
            for g in range(BPW // L):
                p = t * BPW + g * L
                v = nbr_raw[p // NP, pl.ds(p % NP, L)]
                nbr2d[t, pl.ds(g * L, L)] = jnp.where(v < 0, N, v)

        scope_f.__exit__(None, None, None)
        scope_g = scope("nsgather"); scope_g.__enter__()
        ns_copies = []
        for t in range(NROW):
            ns_copies.append(pltpu.async_copy(
                slkp_h.at[nbr2d.at[t]],
                ns_flat.at[pl.ds(t * BPW, BPW)], sem_s))

        for cp in ns_copies:
            cp.wait()
        cw.wait()
        cl.wait()
        cs.wait()
        scope_g.__exit__(None, None, None)
        scope_c = scope("quadform"); scope_c.__enter__()

        iota = lax.broadcasted_iota(jnp.int32, (L,), 0)

        def group_body(g, carry):
            e = g * L
            erow = e + iota
            nsbase = erow * NP
            cols = [s_v[pl.ds(e, L)]]
            for i in range(1, NPTS):
                cols.append(plsc.load_gather(ns_flat, [nsbase + i]))
            s = cols[0]
            for i in range(1, NPTS):
                s = s + cols[i]
            mean = s * (1.0 / NPTS)
            X = [c - mean for c in cols]
            wbase = erow * NSQ
            m = jnp.zeros((L,), jnp.float32)
            for i in range(NPTS):
                y = jnp.zeros((L,), jnp.float32)
                for j in range(NPTS):
                    wv = plsc.load_gather(w_flat, [wbase + (i * NPTS + j)])
                    y = y + wv * X[j]
                m = m + X[i] * y
            m_v[pl.ds(e, L)] = m
            return carry

        lax.fori_loop(0, NGR, group_body, 0)
        scope_c.__exit__(None, None, None)

        pltpu.sync_copy(m_v, m_out.at[pl.ds(base, BPW)])
        pltpu.sync_copy(locs_v, locs_out.at[pl.ds(base, BPW)])
        pltpu.sync_copy(scales_v, scales_out.at[pl.ds(base, BPW)])

    return sc_kernel


def _make_w_gather_tc(B, N, NPTS):
    BLK = 128
    NB = B // BLK

    def body(idx_ref, w_any, o_ref, sem):
        g = pl.program_id(0)
        cps = []
        for u in range(BLK):
            iv = idx_ref[g * BLK + u]
            cp = pltpu.make_async_copy(w_any.at[iv], o_ref.at[u], sem)
            cp.start()
            cps.append(cp)
        for cp in cps:
            cp.wait()

    grid_spec = pltpu.PrefetchScalarGridSpec(
        num_scalar_prefetch=1,
        grid=(NB,),
        in_specs=[pl.BlockSpec(memory_space=pl.ANY)],
        out_specs=pl.BlockSpec((BLK, NPTS, NPTS), lambda g, idx_ref: (g, 0, 0)),
        scratch_shapes=[pltpu.SemaphoreType.DMA],
    )
    return pl.pallas_call(
        body,
        grid_spec=grid_spec,
        out_shape=jax.ShapeDtypeStruct((B, NPTS, NPTS), jnp.float32),
    )


def _tc_tail_body(m_ref, locs_ref, scales_ref, o_ref):
    m = m_ref[...]
    locs = locs_ref[...]
    scales = scales_ref[...]
    lmin = jnp.min(locs)
    left = jnp.minimum(m, 2.0 * lmin - m)
    z = jnp.clip((left - locs) / scales, -5.0, 0.0)
    cdf2 = 1.0 + lax.erf(z * (1.0 / math.sqrt(2.0)))
    o_ref[...] = -jnp.log(cdf2 + 1e-12)


@jax.jit
def kernel(idx, scores, scores_lookup, neighborhood_scores_idx_lookup,
           weight_matrix_lookup, mean_lookup, std_lookup):
    B = idx.shape[0]
    N = scores_lookup.shape[0] - 1
    NPTS = neighborhood_scores_idx_lookup.shape[1]

    nbr_pad = jnp.pad(neighborhood_scores_idx_lookup,
                      ((0, 0), (0, NP - NPTS))).reshape(N * NP)
    w_g = _make_w_gather_tc(B, N, NPTS)(idx, weight_matrix_lookup)
    w_rows = w_g.reshape(B * NPTS * NPTS)

    sc_k = _make_sc_kernel(B, N, NPTS)
    m, locs, scales = sc_k(idx, scores, scores_lookup, nbr_pad, w_rows,
                           mean_lookup, std_lookup)

    rows = B // 128
    out = pl.pallas_call(
        _tc_tail_body,
        out_shape=jax.ShapeDtypeStruct((rows, 128), jnp.float32),
    )(m.reshape(rows, 128), locs.reshape(rows, 128), scales.reshape(rows, 128))
    return out.reshape(B)


# speedup vs baseline: 2.2616x; 1.0014x over previous
import functools
import math

import jax
import jax.numpy as jnp
from jax import lax
from jax.experimental import pallas as pl
from jax.experimental.pallas import tpu as pltpu
from jax.experimental.pallas import tpu_sc as plsc

NC = 2
NS = 16
L = 16
NP = 32


def _make_sc_kernel(B, N, NPTS):
    NW = NC * NS
    BPW = B // NW
    NGR = BPW // L
    FLAT = BPW * NP
    NSQ = NPTS * NPTS
    NROW = FLAT // BPW

    mesh = plsc.VectorSubcoreMesh(core_axis_name="c", subcore_axis_name="s",
                                  num_cores=NC, num_subcores=NS)

    @functools.partial(
        pl.kernel,
        out_type=(
            jax.ShapeDtypeStruct((B,), jnp.float32),
            jax.ShapeDtypeStruct((B,), jnp.float32),
            jax.ShapeDtypeStruct((B,), jnp.float32),
        ),
        mesh=mesh,
        compiler_params=pltpu.CompilerParams(needs_layout_passes=False,
                                             use_tc_tiling_on_sc=False),
        scratch_types=[
            pltpu.VMEM((BPW,), jnp.int32),
            pltpu.VMEM((BPW, NP), jnp.int32),
            pltpu.VMEM((NROW, BPW), jnp.int32),
            pltpu.VMEM((FLAT,), jnp.float32),
            pltpu.VMEM((BPW * NSQ,), jnp.float32),
            pltpu.VMEM((BPW,), jnp.float32),
            pltpu.VMEM((BPW,), jnp.float32),
            pltpu.VMEM((BPW,), jnp.float32),
            pltpu.VMEM((BPW,), jnp.float32),
            pltpu.SemaphoreType.DMA,
            pltpu.SemaphoreType.DMA,
            pltpu.SemaphoreType.DMA,
            pltpu.SemaphoreType.DMA,
        ],
    )
    def sc_kernel(idx_h, scores_h, slkp_h, nbr_h, w_h, mean_h, std_h,
                  m_out, locs_out, scales_out,
                  idx_v, nbr_raw, nbr2d, ns_flat, w_flat, s_v, locs_v,
                  scales_v, m_v, sem_w, sem_n, sem_ls, sem_s):
        wid = lax.axis_index("s") * NC + lax.axis_index("c")
        base = wid * BPW

        pltpu.sync_copy(idx_h.at[pl.ds(base, BPW)], idx_v)
        cl = pltpu.async_copy(mean_h.at[idx_v], locs_v, sem_ls)
        cs = pltpu.async_copy(std_h.at[idx_v], scales_v, sem_ls)
        pltpu.sync_copy(scores_h.at[pl.ds(base, BPW)], s_v)

        cw = pltpu.async_copy(w_h.at[pl.ds(base * NSQ, BPW * NSQ)], w_flat,
                              sem_w)
        for c in range(BPW // L):
            ivec = idx_v[pl.ds(c * L, L)]
            for u in range(L):
                e = c * L + u
                pltpu.make_async_copy(nbr_h.at[pl.ds(ivec[u] * NP, NP)],
                                      nbr_raw.at[e], sem_n).start()

        def nbr_drain(e, carry):
            pltpu.make_async_copy(nbr_h.at[pl.ds(0, NP)], nbr_raw.at[0],
                                  sem_n).wait()
            return carry

        lax.fori_loop(0, BPW, nbr_drain, 0)
        def fix_body(k, carry):
            p = k * L
            v = nbr_raw[p // NP, pl.ds(p % NP, L)]
            nbr2d[p // BPW, pl.ds(p % BPW, L)] = jnp.where(v < 0, N, v)
            return carry

        lax.fori_loop(0, FLAT // L, fix_body, 0)

        ns_copies = []
        for t in range(NROW):
            ns_copies.append(pltpu.async_copy(
                slkp_h.at[nbr2d.at[t]],
                ns_flat.at[pl.ds(t * BPW, BPW)], sem_s))

        for cp in ns_copies:
            cp.wait()
        cw.wait()
        cl.wait()
        cs.wait()

        iota = lax.broadcasted_iota(jnp.int32, (L,), 0)

        def group_body(g, carry):
            e = g * L
            erow = e + iota
            nsbase = erow * NP
            cols = [s_v[pl.ds(e, L)]]
            for i in range(1, NPTS):
                cols.append(plsc.load_gather(ns_flat, [nsbase + i]))
            s = cols[0]
            for i in range(1, NPTS):
                s = s + cols[i]
            mean = s * (1.0 / NPTS)
            X = [c - mean for c in cols]
            wbase = erow * NSQ
            m = jnp.zeros((L,), jnp.float32)
            for i in range(NPTS):
                y = jnp.zeros((L,), jnp.float32)
                for j in range(NPTS):
                    wv = plsc.load_gather(w_flat, [wbase + (i * NPTS + j)])
                    y = y + wv * X[j]
                m = m + X[i] * y
            m_v[pl.ds(e, L)] = m
            return carry

        lax.fori_loop(0, NGR, group_body, 0)

        pltpu.sync_copy(m_v, m_out.at[pl.ds(base, BPW)])
        pltpu.sync_copy(locs_v, locs_out.at[pl.ds(base, BPW)])
        pltpu.sync_copy(scales_v, scales_out.at[pl.ds(base, BPW)])

    return sc_kernel


def _make_w_gather_tc(B, N, NPTS):
    BLK = 128
    NB = B // BLK

    def body(idx_ref, w_any, o_ref, sem):
        g = pl.program_id(0)
        cps = []
        for u in range(BLK):
            iv = idx_ref[g * BLK + u]
            cp = pltpu.make_async_copy(w_any.at[iv], o_ref.at[u], sem)
            cp.start()
            cps.append(cp)
        for cp in cps:
            cp.wait()

    grid_spec = pltpu.PrefetchScalarGridSpec(
        num_scalar_prefetch=1,
        grid=(NB,),
        in_specs=[pl.BlockSpec(memory_space=pl.ANY)],
        out_specs=pl.BlockSpec((BLK, NPTS, NPTS), lambda g, idx_ref: (g, 0, 0)),
        scratch_shapes=[pltpu.SemaphoreType.DMA],
    )
    return pl.pallas_call(
        body,
        grid_spec=grid_spec,
        out_shape=jax.ShapeDtypeStruct((B, NPTS, NPTS), jnp.float32),
    )


def _tc_tail_body(m_ref, locs_ref, scales_ref, o_ref):
    m = m_ref[...]
    locs = locs_ref[...]
    scales = scales_ref[...]
    lmin = jnp.min(locs)
    left = jnp.minimum(m, 2.0 * lmin - m)
    z = jnp.clip((left - locs) / scales, -5.0, 0.0)
    cdf2 = 1.0 + lax.erf(z * (1.0 / math.sqrt(2.0)))
    o_ref[...] = -jnp.log(cdf2 + 1e-12)


@jax.jit
def kernel(idx, scores, scores_lookup, neighborhood_scores_idx_lookup,
           weight_matrix_lookup, mean_lookup, std_lookup):
    B = idx.shape[0]
    N = scores_lookup.shape[0] - 1
    NPTS = neighborhood_scores_idx_lookup.shape[1]

    nbr_pad = jnp.pad(neighborhood_scores_idx_lookup,
                      ((0, 0), (0, NP - NPTS))).reshape(N * NP)
    w_g = _make_w_gather_tc(B, N, NPTS)(idx, weight_matrix_lookup)
    w_rows = w_g.reshape(B * NPTS * NPTS)

    sc_k = _make_sc_kernel(B, N, NPTS)
    m, locs, scales = sc_k(idx, scores, scores_lookup, nbr_pad, w_rows,
                           mean_lookup, std_lookup)

    rows = B // 128
    out = pl.pallas_call(
        _tc_tail_body,
        out_shape=jax.ShapeDtypeStruct((rows, 128), jnp.float32),
    )(m.reshape(rows, 128), locs.reshape(rows, 128), scales.reshape(rows, 128))
    return out.reshape(B)
